# Initial kernel scaffold; baseline (speedup 1.0000x reference)
import jax
import jax.numpy as jnp
from jax.experimental import pallas as pl


def kernel(hidden_states, gate_w, e_score_correction_bias, w_gate_up, w_down, shared_gate_up, shared_down):
    raise NotImplementedError("write your pallas kernel here")


# speedup vs baseline: 1.4972x; 1.4972x over previous
import functools

import jax
import jax.numpy as jnp
from jax import lax
from jax.experimental import pallas as pl
from jax.experimental.pallas import tpu as pltpu
from jax.experimental.pallas import tpu_sc as plsc

T, H, E, I = 2048, 2048, 8, 1024
ISH = 2048
TOPK = 2
N_GROUP = 4
SCALE = 2.5

BT = 256
NT = 24
R = NT * BT
NW = 32
PW = T * TOPK // NW
RS = R // NW
GCH = 16
TPW = T // NW


def _router_body(x_ref, gw_ref, bias_ref, ids_ref, wts_ref):
    x = x_ref[...].astype(jnp.bfloat16)
    gw = gw_ref[...].astype(jnp.bfloat16)
    logits = lax.dot_general(
        x, gw, (((1,), (0,)), ((), ())),
        preferred_element_type=jnp.float32)
    scores = jax.nn.sigmoid(logits)
    sc = scores + bias_ref[0:1, :]
    bt = sc.shape[0]
    iota8 = lax.broadcasted_iota(jnp.int32, (bt, E), 1)
    gi = iota8 // (E // N_GROUP)

    r8 = lax.broadcasted_iota(jnp.int32, (E, E), 0)
    c8 = lax.broadcasted_iota(jnp.int32, (E, E), 1)
    pairm = (r8 // 2 == c8 // 2).astype(jnp.float32)
    gs8 = lax.dot_general(
        sc, pairm, (((1,), (0,)), ((), ())),
        precision=lax.Precision.HIGHEST,
        preferred_element_type=jnp.float32)

    m1 = jnp.max(gs8, axis=1, keepdims=True)
    g1 = jnp.min(jnp.where(gs8 == m1, gi, N_GROUP), axis=1, keepdims=True)
    gs8b = jnp.where(gi == g1, -jnp.inf, gs8)
    m2 = jnp.max(gs8b, axis=1, keepdims=True)
    g2 = jnp.min(jnp.where(gs8b == m2, gi, N_GROUP), axis=1, keepdims=True)
    sel = (gi == g1) | (gi == g2)

    tmp = jnp.where(sel, sc, 0.0)
    t1 = jnp.max(tmp, axis=1, keepdims=True)
    e1 = jnp.min(jnp.where(tmp == t1, iota8, E), axis=1, keepdims=True)
    tmp2 = jnp.where(iota8 == e1, -jnp.inf, tmp)
    t2 = jnp.max(tmp2, axis=1, keepdims=True)
    e2 = jnp.min(jnp.where(tmp2 == t2, iota8, E), axis=1, keepdims=True)

    w1 = jnp.sum(jnp.where(iota8 == e1, scores, 0.0), axis=1, keepdims=True)
    w2 = jnp.sum(jnp.where(iota8 == e2, scores, 0.0), axis=1, keepdims=True)
    denom = w1 + w2 + 1e-20
    ids_ref[...] = jnp.concatenate([e1, e2], axis=1)
    wts_ref[...] = jnp.concatenate([w1, w2], axis=1) * (SCALE / denom)


def _count_body(ids_hbm, cnts_hbm, ids_v, cnt_v):
    wid = lax.axis_index("s") * 2 + lax.axis_index("c")
    iota = lax.iota(jnp.int32, 16)
    pltpu.sync_copy(ids_hbm.at[pl.ds(wid * PW, PW)], ids_v)
    cnt = jnp.zeros(16, jnp.int32)
    for v in range(PW // 16):
        vec = ids_v[pl.ds(v * 16, 16)]
        for e in range(E):
            pop = jnp.sum((vec == e).astype(jnp.int32))
            cnt = cnt + (iota == e).astype(jnp.int32) * pop
    cnt_v[...] = cnt
    pltpu.sync_copy(cnt_v, cnts_hbm.at[wid])


def _dispatch_body(ids_hbm, cnts_hbm, hid_hbm, xs_hbm, slot_hbm, te_hbm,
                   nact_hbm, ids_v, slots_v, idxc_v, allcnt_v, rowbuf_v,
                   tev_v, nactv_v, sem, lsem):
    ci = lax.axis_index("c")
    si = lax.axis_index("s")
    wid = si * 2 + ci
    base_p = wid * PW
    iota = lax.iota(jnp.int32, 16)

    pltpu.sync_copy(ids_hbm.at[pl.ds(base_p, PW)], ids_v)
    pltpu.sync_copy(cnts_hbm, allcnt_v)

    tot = jnp.zeros(16, jnp.int32)
    pref = jnp.zeros(16, jnp.int32)
    for ww in range(NW):
        row = allcnt_v[ww, :]
        tot = tot + row
        pref = pref + row * (ww < wid).astype(jnp.int32)
    totpad = (tot + (BT - 1)) // BT * BT
    incl = plsc.cumsum(totpad)
    excl = incl - totpad
    basev = excl + pref
    base_sc = [jnp.sum(basev * (iota == e).astype(jnp.int32)) for e in range(E)]
    incl_sc = [jnp.sum(incl * (iota == e).astype(jnp.int32)) for e in range(E)]
    total_pad = incl_sc[E - 1]

    run = list(base_sc)
    for v in range(PW // 16):
        vec = ids_v[pl.ds(v * 16, 16)]
        slot_vec = jnp.zeros(16, jnp.int32)
        for e in range(E):
            m = vec == e
            mi = m.astype(jnp.int32)
            ranks = plsc.cumsum(mi) - 1
            slot_vec = jnp.where(m, run[e] + ranks, slot_vec)
            run[e] = run[e] + jnp.sum(mi)
        slots_v[pl.ds(v * 16, 16)] = slot_vec
    pltpu.sync_copy(slots_v, slot_hbm.at[pl.ds(base_p, PW)])

    @pl.when(wid == 0)
    def _():
        for half in range(2):
            ivec = (iota + 16 * half) * BT
            acc = jnp.zeros(16, jnp.int32)
            for e in range(E):
                acc = acc + (ivec >= incl_sc[e]).astype(jnp.int32)
            tev_v[pl.ds(16 * half, 16)] = jnp.minimum(acc, E - 1)
        nactv_v[...] = (iota == 0).astype(jnp.int32) * (total_pad // BT)
        pltpu.sync_copy(tev_v, te_hbm)
        pltpu.sync_copy(nactv_v, nact_hbm)

    tok0 = wid * PW % T
    nch = PW // GCH
    lcp = [None, None]
    scp = [None, None]
    lcp[0] = pltpu.async_copy(hid_hbm.at[pl.ds(tok0, GCH)], rowbuf_v.at[0], lsem)
    for ch in range(nch):
        b = ch % 2
        lcp[b].wait()
        for i in range(GCH // 16):
            idxc_v[b, pl.ds(i * 16, 16)] = slots_v[pl.ds(ch * GCH + i * 16, 16)]
        scp[b] = pltpu.async_copy(rowbuf_v.at[b], xs_hbm.at[idxc_v.at[b]], sem)
        if ch + 1 < nch:
            if scp[1 - b] is not None:
                scp[1 - b].wait()
            lcp[1 - b] = pltpu.async_copy(
                hid_hbm.at[pl.ds(tok0 + (ch + 1) * GCH, GCH)],
                rowbuf_v.at[1 - b], lsem)
    for b in range(2):
        if scp[b] is None:
            continue
        scp[b].wait()


def _gemm_body(te_ref, nact_ref, xs_ref, wgu_ref, wd_ref, ys_ref):
    i = pl.program_id(0)

    @pl.when(i < nact_ref[0])
    def _():
        xb = xs_ref[...].astype(jnp.bfloat16)
        h1 = lax.dot_general(
            xb, wgu_ref[0], (((1,), (0,)), ((), ())),
            preferred_element_type=jnp.float32)
        g = h1[:, :I]
        u = h1[:, I:]
        act = (jax.nn.silu(g) * u).astype(jnp.bfloat16)
        ys_ref[...] = lax.dot_general(
            act, wd_ref[0], (((1,), (0,)), ((), ())),
            preferred_element_type=jnp.float32)


def _shared_body(x_ref, wgu_ref, wd_ref, y0_ref, y1_ref, wts_ref, out_ref):
    xb = x_ref[...].astype(jnp.bfloat16)
    gu = lax.dot_general(
        xb, wgu_ref[...], (((1,), (0,)), ((), ())),
        preferred_element_type=jnp.float32)
    g = gu[:, :ISH]
    u = gu[:, ISH:]
    act = (jax.nn.silu(g) * u).astype(jnp.bfloat16)
    s = lax.dot_general(
        act, wd_ref[...], (((1,), (0,)), ((), ())),
        preferred_element_type=jnp.float32)
    wts = wts_ref[...]
    iota2 = lax.broadcasted_iota(jnp.int32, (BT, TOPK), 1)
    w0 = jnp.sum(jnp.where(iota2 == 0, wts, 0.0), axis=1, keepdims=True)
    w1 = jnp.sum(jnp.where(iota2 == 1, wts, 0.0), axis=1, keepdims=True)
    out_ref[...] = s + w0 * y0_ref[...] + w1 * y1_ref[...]


def _gather_body(ys_hbm, slot_hbm, y0_hbm, y1_hbm, slots_v, idxc_v, rows_v,
                 sem, lsem):
    ci = lax.axis_index("c")
    si = lax.axis_index("s")
    wid = si * 2 + ci
    pltpu.sync_copy(slot_hbm.at[pl.ds(wid * TPW, TPW)],
                    slots_v.at[pl.ds(0, TPW)])
    pltpu.sync_copy(slot_hbm.at[pl.ds(T + wid * TPW, TPW)],
                    slots_v.at[pl.ds(TPW, TPW)])
    nch = TPW // 16
    steps = [(k, ch) for k in range(2) for ch in range(nch)]
    gcp = [None, None]
    wcp = [None, None]

    def start_gather(i, b):
        k, ch = steps[i]
        idxc_v[b, pl.ds(0, 16)] = slots_v[pl.ds(k * TPW + ch * 16, 16)]
        return pltpu.async_copy(ys_hbm.at[idxc_v.at[b]], rows_v.at[b], lsem)

    gcp[0] = start_gather(0, 0)
    for i in range(len(steps)):
        b = i % 2
        k, ch = steps[i]
        gcp[b].wait()
        dst = y0_hbm if k == 0 else y1_hbm
        wcp[b] = pltpu.async_copy(
            rows_v.at[b],
            dst.at[pl.ds(wid * TPW + ch * 16, 16)],
            sem)
        if i + 1 < len(steps):
            if wcp[1 - b] is not None:
                wcp[1 - b].wait()
            gcp[1 - b] = start_gather(i + 1, 1 - b)
    for b in range(2):
        if wcp[b] is None:
            continue
        wcp[b].wait()


def kernel(hidden_states, gate_w, e_score_correction_bias, w_gate_up, w_down,
           shared_gate_up, shared_down):
    bias8 = jnp.broadcast_to(e_score_correction_bias[None, :], (8, E))

    ids, wts = pl.pallas_call(
        _router_body,
        grid=(T // BT,),
        in_specs=[
            pl.BlockSpec((BT, H), lambda t: (t, 0)),
            pl.BlockSpec((H, E), lambda t: (0, 0)),
            pl.BlockSpec((8, E), lambda t: (0, 0)),
        ],
        out_specs=[
            pl.BlockSpec((BT, TOPK), lambda t: (t, 0)),
            pl.BlockSpec((BT, TOPK), lambda t: (t, 0)),
        ],
        out_shape=[
            jax.ShapeDtypeStruct((T, TOPK), jnp.int32),
            jax.ShapeDtypeStruct((T, TOPK), jnp.float32),
        ],
    )(hidden_states, gate_w, bias8)

    ids_flat = ids.T.reshape(T * TOPK)

    mesh = plsc.VectorSubcoreMesh(core_axis_name="c", subcore_axis_name="s")
    cnts = pl.kernel(
        _count_body,
        mesh=mesh,
        compiler_params=pltpu.CompilerParams(needs_layout_passes=False),
        out_type=jax.ShapeDtypeStruct((NW, 16), jnp.int32),
        scratch_types=[
            pltpu.VMEM((PW,), jnp.int32),
            pltpu.VMEM((16,), jnp.int32),
        ],
    )(ids_flat)

    xs, slot_arr, te, nact = pl.kernel(
        _dispatch_body,
        mesh=plsc.VectorSubcoreMesh(core_axis_name="c", subcore_axis_name="s"),
        compiler_params=pltpu.CompilerParams(needs_layout_passes=False),
        out_type=(
            jax.ShapeDtypeStruct((R, H), jnp.float32),
            jax.ShapeDtypeStruct((T * TOPK,), jnp.int32),
            jax.ShapeDtypeStruct((NW,), jnp.int32),
            jax.ShapeDtypeStruct((16,), jnp.int32),
        ),
        scratch_types=[
            pltpu.VMEM((PW,), jnp.int32),
            pltpu.VMEM((PW,), jnp.int32),
            pltpu.VMEM((2, GCH), jnp.int32),
            pltpu.VMEM((NW, 16), jnp.int32),
            pltpu.VMEM((2, GCH, H), jnp.float32),
            pltpu.VMEM((NW,), jnp.int32),
            pltpu.VMEM((16,), jnp.int32),
            pltpu.SemaphoreType.DMA,
            pltpu.SemaphoreType.DMA,
        ],
    )(ids_flat, cnts, hidden_states)

    wgu_bf = w_gate_up.astype(jnp.bfloat16)
    wd_bf = w_down.astype(jnp.bfloat16)
    ys = pl.pallas_call(
        _gemm_body,
        grid_spec=pltpu.PrefetchScalarGridSpec(
            num_scalar_prefetch=2,
            grid=(NT,),
            in_specs=[
                pl.BlockSpec((BT, H), lambda i, te, na: (i, 0)),
                pl.BlockSpec((1, H, 2 * I), lambda i, te, na: (te[i], 0, 0)),
                pl.BlockSpec((1, I, H), lambda i, te, na: (te[i], 0, 0)),
            ],
            out_specs=pl.BlockSpec((BT, H), lambda i, te, na: (i, 0)),
        ),
        out_shape=jax.ShapeDtypeStruct((R, H), jnp.float32),
    )(te, nact, xs, wgu_bf, wd_bf)

    y0, y1 = pl.kernel(
        _gather_body,
        mesh=plsc.VectorSubcoreMesh(core_axis_name="c", subcore_axis_name="s"),
        compiler_params=pltpu.CompilerParams(needs_layout_passes=False),
        out_type=(
            jax.ShapeDtypeStruct((T, H), jnp.float32),
            jax.ShapeDtypeStruct((T, H), jnp.float32),
        ),
        scratch_types=[
            pltpu.VMEM((PW,), jnp.int32),
            pltpu.VMEM((2, 16), jnp.int32),
            pltpu.VMEM((2, 16, H), jnp.float32),
            pltpu.SemaphoreType.DMA,
            pltpu.SemaphoreType.DMA,
        ],
    )(ys, slot_arr)

    sgu_bf = shared_gate_up.astype(jnp.bfloat16)
    sd_bf = shared_down.astype(jnp.bfloat16)
    out = pl.pallas_call(
        _shared_body,
        grid=(T // BT,),
        in_specs=[
            pl.BlockSpec((BT, H), lambda t: (t, 0)),
            pl.BlockSpec((H, 2 * ISH), lambda t: (0, 0)),
            pl.BlockSpec((ISH, H), lambda t: (0, 0)),
            pl.BlockSpec((BT, H), lambda t: (t, 0)),
            pl.BlockSpec((BT, H), lambda t: (t, 0)),
            pl.BlockSpec((BT, TOPK), lambda t: (t, 0)),
        ],
        out_specs=pl.BlockSpec((BT, H), lambda t: (t, 0)),
        out_shape=jax.ShapeDtypeStruct((T, H), jnp.float32),
    )(hidden_states, sgu_bf, sd_bf, y0, y1, wts)
    return out


# speedup vs baseline: 1.6173x; 1.0803x over previous
import functools

import jax
import jax.numpy as jnp
from jax import lax
from jax.experimental import pallas as pl
from jax.experimental.pallas import tpu as pltpu
from jax.experimental.pallas import tpu_sc as plsc

T, H, E, I = 2048, 2048, 8, 1024
ISH = 2048
TOPK = 2
N_GROUP = 4
SCALE = 2.5

BT = 256
NT = 24
R = NT * BT
NW = 32
PW = T * TOPK // NW
RS = R // NW
GCH = 16
TPW = T // NW
H2 = H // 2
MASK_HI = -65536


def _pack_bf16(x):
    lo = x[:, :H2].astype(jnp.bfloat16).astype(jnp.float32)
    hi = x[:, H2:].astype(jnp.bfloat16).astype(jnp.float32)
    lob = lax.bitcast_convert_type(lo, jnp.int32)
    hib = lax.bitcast_convert_type(hi, jnp.int32)
    return jnp.bitwise_or(jnp.bitwise_and(hib, MASK_HI),
                          jnp.bitwise_and(lax.shift_right_logical(lob, 16),
                                          65535))


def _unpack_bf16(p):
    lo = lax.bitcast_convert_type(lax.shift_left(p, 16),
                                  jnp.float32).astype(jnp.bfloat16)
    hi = lax.bitcast_convert_type(jnp.bitwise_and(p, MASK_HI),
                                  jnp.float32).astype(jnp.bfloat16)
    return lo, hi


def _router_body(x_ref, gw_ref, bias_ref, ids_ref, wts_ref, xp_ref):
    xp_ref[...] = _pack_bf16(x_ref[...])
    x = x_ref[...].astype(jnp.bfloat16)
    gw = gw_ref[...].astype(jnp.bfloat16)
    logits = lax.dot_general(
        x, gw, (((1,), (0,)), ((), ())),
        preferred_element_type=jnp.float32)
    scores = jax.nn.sigmoid(logits)
    sc = scores + bias_ref[0:1, :]
    bt = sc.shape[0]
    iota8 = lax.broadcasted_iota(jnp.int32, (bt, E), 1)
    gi = iota8 // (E // N_GROUP)

    r8 = lax.broadcasted_iota(jnp.int32, (E, E), 0)
    c8 = lax.broadcasted_iota(jnp.int32, (E, E), 1)
    pairm = (r8 // 2 == c8 // 2).astype(jnp.float32)
    gs8 = lax.dot_general(
        sc, pairm, (((1,), (0,)), ((), ())),
        precision=lax.Precision.HIGHEST,
        preferred_element_type=jnp.float32)

    m1 = jnp.max(gs8, axis=1, keepdims=True)
    g1 = jnp.min(jnp.where(gs8 == m1, gi, N_GROUP), axis=1, keepdims=True)
    gs8b = jnp.where(gi == g1, -jnp.inf, gs8)
    m2 = jnp.max(gs8b, axis=1, keepdims=True)
    g2 = jnp.min(jnp.where(gs8b == m2, gi, N_GROUP), axis=1, keepdims=True)
    sel = (gi == g1) | (gi == g2)

    tmp = jnp.where(sel, sc, 0.0)
    t1 = jnp.max(tmp, axis=1, keepdims=True)
    e1 = jnp.min(jnp.where(tmp == t1, iota8, E), axis=1, keepdims=True)
    tmp2 = jnp.where(iota8 == e1, -jnp.inf, tmp)
    t2 = jnp.max(tmp2, axis=1, keepdims=True)
    e2 = jnp.min(jnp.where(tmp2 == t2, iota8, E), axis=1, keepdims=True)

    w1 = jnp.sum(jnp.where(iota8 == e1, scores, 0.0), axis=1, keepdims=True)
    w2 = jnp.sum(jnp.where(iota8 == e2, scores, 0.0), axis=1, keepdims=True)
    denom = w1 + w2 + 1e-20
    ids_ref[...] = jnp.concatenate([e1, e2], axis=1)
    wts_ref[...] = jnp.concatenate([w1, w2], axis=1) * (SCALE / denom)


def _count_body(ids_hbm, cnts_hbm, ids_v, cnt_v):
    wid = lax.axis_index("s") * 2 + lax.axis_index("c")
    iota = lax.iota(jnp.int32, 16)
    pltpu.sync_copy(ids_hbm.at[pl.ds(wid * PW, PW)], ids_v)
    cnt = jnp.zeros(16, jnp.int32)
    for v in range(PW // 16):
        vec = ids_v[pl.ds(v * 16, 16)]
        for e in range(E):
            pop = jnp.sum((vec == e).astype(jnp.int32))
            cnt = cnt + (iota == e).astype(jnp.int32) * pop
    cnt_v[...] = cnt
    pltpu.sync_copy(cnt_v, cnts_hbm.at[wid])


def _dispatch_body(ids_hbm, cnts_hbm, hid_hbm, xs_hbm, slot_hbm, te_hbm,
                   nact_hbm, ids_v, slots_v, idxc_v, allcnt_v, rowbuf_v,
                   tev_v, nactv_v, sem, lsem):
    ci = lax.axis_index("c")
    si = lax.axis_index("s")
    wid = si * 2 + ci
    base_p = wid * PW
    iota = lax.iota(jnp.int32, 16)

    pltpu.sync_copy(ids_hbm.at[pl.ds(base_p, PW)], ids_v)
    pltpu.sync_copy(cnts_hbm, allcnt_v)

    tot = jnp.zeros(16, jnp.int32)
    pref = jnp.zeros(16, jnp.int32)
    for ww in range(NW):
        row = allcnt_v[ww, :]
        tot = tot + row
        pref = pref + row * (ww < wid).astype(jnp.int32)
    totpad = (tot + (BT - 1)) // BT * BT
    incl = plsc.cumsum(totpad)
    excl = incl - totpad
    basev = excl + pref
    base_sc = [jnp.sum(basev * (iota == e).astype(jnp.int32)) for e in range(E)]
    incl_sc = [jnp.sum(incl * (iota == e).astype(jnp.int32)) for e in range(E)]
    total_pad = incl_sc[E - 1]

    run = list(base_sc)
    for v in range(PW // 16):
        vec = ids_v[pl.ds(v * 16, 16)]
        slot_vec = jnp.zeros(16, jnp.int32)
        for e in range(E):
            m = vec == e
            mi = m.astype(jnp.int32)
            ranks = plsc.cumsum(mi) - 1
            slot_vec = jnp.where(m, run[e] + ranks, slot_vec)
            run[e] = run[e] + jnp.sum(mi)
        slots_v[pl.ds(v * 16, 16)] = slot_vec
    pltpu.sync_copy(slots_v, slot_hbm.at[pl.ds(base_p, PW)])

    @pl.when(wid == 0)
    def _():
        for half in range(2):
            ivec = (iota + 16 * half) * BT
            acc = jnp.zeros(16, jnp.int32)
            for e in range(E):
                acc = acc + (ivec >= incl_sc[e]).astype(jnp.int32)
            tev_v[pl.ds(16 * half, 16)] = jnp.minimum(acc, E - 1)
        nactv_v[...] = (iota == 0).astype(jnp.int32) * (total_pad // BT)
        pltpu.sync_copy(tev_v, te_hbm)
        pltpu.sync_copy(nactv_v, nact_hbm)

    tok0 = wid * PW % T
    nch = PW // GCH
    lcp = [None, None]
    scp = [None, None]
    lcp[0] = pltpu.async_copy(hid_hbm.at[pl.ds(tok0, GCH)], rowbuf_v.at[0], lsem)
    for ch in range(nch):
        b = ch % 2
        lcp[b].wait()
        for i in range(GCH // 16):
            idxc_v[b, pl.ds(i * 16, 16)] = slots_v[pl.ds(ch * GCH + i * 16, 16)]
        scp[b] = pltpu.async_copy(rowbuf_v.at[b], xs_hbm.at[idxc_v.at[b]], sem)
        if ch + 1 < nch:
            if scp[1 - b] is not None:
                scp[1 - b].wait()
            lcp[1 - b] = pltpu.async_copy(
                hid_hbm.at[pl.ds(tok0 + (ch + 1) * GCH, GCH)],
                rowbuf_v.at[1 - b], lsem)
    for b in range(2):
        if scp[b] is None:
            continue
        scp[b].wait()


def _gemm_body(te_ref, nact_ref, xs_ref, wgu_ref, wd_ref, ys_ref):
    i = pl.program_id(0)

    @pl.when(i < nact_ref[0])
    def _():
        xlo, xhi = _unpack_bf16(xs_ref[...])
        wgu = wgu_ref[0]
        h1 = lax.dot_general(
            xlo, wgu[:H2, :], (((1,), (0,)), ((), ())),
            preferred_element_type=jnp.float32)
        h1 = h1 + lax.dot_general(
            xhi, wgu[H2:, :], (((1,), (0,)), ((), ())),
            preferred_element_type=jnp.float32)
        g = h1[:, :I]
        u = h1[:, I:]
        act = (jax.nn.silu(g) * u).astype(jnp.bfloat16)
        ys_ref[...] = _pack_bf16(lax.dot_general(
            act, wd_ref[0], (((1,), (0,)), ((), ())),
            preferred_element_type=jnp.float32))


def _shared_body(x_ref, wgu_ref, wd_ref, y0_ref, y1_ref, wts_ref, out_ref):
    xb = x_ref[...].astype(jnp.bfloat16)
    gu = lax.dot_general(
        xb, wgu_ref[...], (((1,), (0,)), ((), ())),
        preferred_element_type=jnp.float32)
    g = gu[:, :ISH]
    u = gu[:, ISH:]
    act = (jax.nn.silu(g) * u).astype(jnp.bfloat16)
    s = lax.dot_general(
        act, wd_ref[...], (((1,), (0,)), ((), ())),
        preferred_element_type=jnp.float32)
    wts = wts_ref[...]
    iota2 = lax.broadcasted_iota(jnp.int32, (BT, TOPK), 1)
    w0 = jnp.sum(jnp.where(iota2 == 0, wts, 0.0), axis=1, keepdims=True)
    w1 = jnp.sum(jnp.where(iota2 == 1, wts, 0.0), axis=1, keepdims=True)
    y0lo, y0hi = _unpack_bf16(y0_ref[...])
    y1lo, y1hi = _unpack_bf16(y1_ref[...])
    out_ref[:, :H2] = (s[:, :H2] + w0 * y0lo.astype(jnp.float32)
                       + w1 * y1lo.astype(jnp.float32))
    out_ref[:, H2:] = (s[:, H2:] + w0 * y0hi.astype(jnp.float32)
                       + w1 * y1hi.astype(jnp.float32))


def _gather_body(ys_hbm, slot_hbm, y0_hbm, y1_hbm, slots_v, idxc_v, rows_v,
                 sem, lsem):
    ci = lax.axis_index("c")
    si = lax.axis_index("s")
    wid = si * 2 + ci
    pltpu.sync_copy(slot_hbm.at[pl.ds(wid * TPW, TPW)],
                    slots_v.at[pl.ds(0, TPW)])
    pltpu.sync_copy(slot_hbm.at[pl.ds(T + wid * TPW, TPW)],
                    slots_v.at[pl.ds(TPW, TPW)])
    nch = TPW // 16
    steps = [(k, ch) for k in range(2) for ch in range(nch)]
    gcp = [None, None]
    wcp = [None, None]

    def start_gather(i, b):
        k, ch = steps[i]
        idxc_v[b, pl.ds(0, 16)] = slots_v[pl.ds(k * TPW + ch * 16, 16)]
        return pltpu.async_copy(ys_hbm.at[idxc_v.at[b]], rows_v.at[b], lsem)

    gcp[0] = start_gather(0, 0)
    for i in range(len(steps)):
        b = i % 2
        k, ch = steps[i]
        gcp[b].wait()
        dst = y0_hbm if k == 0 else y1_hbm
        wcp[b] = pltpu.async_copy(
            rows_v.at[b],
            dst.at[pl.ds(wid * TPW + ch * 16, 16)],
            sem)
        if i + 1 < len(steps):
            if wcp[1 - b] is not None:
                wcp[1 - b].wait()
            gcp[1 - b] = start_gather(i + 1, 1 - b)
    for b in range(2):
        if wcp[b] is None:
            continue
        wcp[b].wait()


def kernel(hidden_states, gate_w, e_score_correction_bias, w_gate_up, w_down,
           shared_gate_up, shared_down):
    bias8 = jnp.broadcast_to(e_score_correction_bias[None, :], (8, E))

    ids, wts, xp = pl.pallas_call(
        _router_body,
        grid=(T // BT,),
        in_specs=[
            pl.BlockSpec((BT, H), lambda t: (t, 0)),
            pl.BlockSpec((H, E), lambda t: (0, 0)),
            pl.BlockSpec((8, E), lambda t: (0, 0)),
        ],
        out_specs=[
            pl.BlockSpec((BT, TOPK), lambda t: (t, 0)),
            pl.BlockSpec((BT, TOPK), lambda t: (t, 0)),
            pl.BlockSpec((BT, H2), lambda t: (t, 0)),
        ],
        out_shape=[
            jax.ShapeDtypeStruct((T, TOPK), jnp.int32),
            jax.ShapeDtypeStruct((T, TOPK), jnp.float32),
            jax.ShapeDtypeStruct((T, H2), jnp.int32),
        ],
    )(hidden_states, gate_w, bias8)

    ids_flat = ids.T.reshape(T * TOPK)

    mesh = plsc.VectorSubcoreMesh(core_axis_name="c", subcore_axis_name="s")
    cnts = pl.kernel(
        _count_body,
        mesh=mesh,
        compiler_params=pltpu.CompilerParams(needs_layout_passes=False),
        out_type=jax.ShapeDtypeStruct((NW, 16), jnp.int32),
        scratch_types=[
            pltpu.VMEM((PW,), jnp.int32),
            pltpu.VMEM((16,), jnp.int32),
        ],
    )(ids_flat)

    xs, slot_arr, te, nact = pl.kernel(
        _dispatch_body,
        mesh=plsc.VectorSubcoreMesh(core_axis_name="c", subcore_axis_name="s"),
        compiler_params=pltpu.CompilerParams(needs_layout_passes=False),
        out_type=(
            jax.ShapeDtypeStruct((R, H2), jnp.int32),
            jax.ShapeDtypeStruct((T * TOPK,), jnp.int32),
            jax.ShapeDtypeStruct((NW,), jnp.int32),
            jax.ShapeDtypeStruct((16,), jnp.int32),
        ),
        scratch_types=[
            pltpu.VMEM((PW,), jnp.int32),
            pltpu.VMEM((PW,), jnp.int32),
            pltpu.VMEM((2, GCH), jnp.int32),
            pltpu.VMEM((NW, 16), jnp.int32),
            pltpu.VMEM((2, GCH, H2), jnp.int32),
            pltpu.VMEM((NW,), jnp.int32),
            pltpu.VMEM((16,), jnp.int32),
            pltpu.SemaphoreType.DMA,
            pltpu.SemaphoreType.DMA,
        ],
    )(ids_flat, cnts, xp)

    wgu_bf = w_gate_up.astype(jnp.bfloat16)
    wd_bf = w_down.astype(jnp.bfloat16)
    ys = pl.pallas_call(
        _gemm_body,
        grid_spec=pltpu.PrefetchScalarGridSpec(
            num_scalar_prefetch=2,
            grid=(NT,),
            in_specs=[
                pl.BlockSpec((BT, H2), lambda i, te, na: (i, 0)),
                pl.BlockSpec((1, H, 2 * I), lambda i, te, na: (te[i], 0, 0)),
                pl.BlockSpec((1, I, H), lambda i, te, na: (te[i], 0, 0)),
            ],
            out_specs=pl.BlockSpec((BT, H2), lambda i, te, na: (i, 0)),
        ),
        out_shape=jax.ShapeDtypeStruct((R, H2), jnp.int32),
    )(te, nact, xs, wgu_bf, wd_bf)

    y0, y1 = pl.kernel(
        _gather_body,
        mesh=plsc.VectorSubcoreMesh(core_axis_name="c", subcore_axis_name="s"),
        compiler_params=pltpu.CompilerParams(needs_layout_passes=False),
        out_type=(
            jax.ShapeDtypeStruct((T, H2), jnp.int32),
            jax.ShapeDtypeStruct((T, H2), jnp.int32),
        ),
        scratch_types=[
            pltpu.VMEM((PW,), jnp.int32),
            pltpu.VMEM((2, 16), jnp.int32),
            pltpu.VMEM((2, 16, H2), jnp.int32),
            pltpu.SemaphoreType.DMA,
            pltpu.SemaphoreType.DMA,
        ],
    )(ys, slot_arr)

    sgu_bf = shared_gate_up.astype(jnp.bfloat16)
    sd_bf = shared_down.astype(jnp.bfloat16)
    out = pl.pallas_call(
        _shared_body,
        grid=(T // BT,),
        in_specs=[
            pl.BlockSpec((BT, H), lambda t: (t, 0)),
            pl.BlockSpec((H, 2 * ISH), lambda t: (0, 0)),
            pl.BlockSpec((ISH, H), lambda t: (0, 0)),
            pl.BlockSpec((BT, H2), lambda t: (t, 0)),
            pl.BlockSpec((BT, H2), lambda t: (t, 0)),
            pl.BlockSpec((BT, TOPK), lambda t: (t, 0)),
        ],
        out_specs=pl.BlockSpec((BT, H), lambda t: (t, 0)),
        out_shape=jax.ShapeDtypeStruct((T, H), jnp.float32),
    )(hidden_states, sgu_bf, sd_bf, y0, y1, wts)
    return out


# speedup vs baseline: 1.6342x; 1.0104x over previous
import functools

import jax
import jax.numpy as jnp
from jax import lax
from jax.experimental import pallas as pl
from jax.experimental.pallas import tpu as pltpu
from jax.experimental.pallas import tpu_sc as plsc

T, H, E, I = 2048, 2048, 8, 1024
ISH = 2048
TOPK = 2
N_GROUP = 4
SCALE = 2.5

BT = 256
NT = 24
R = NT * BT
NW = 32
PW = T * TOPK // NW
RS = R // NW
GCH = 16
TPW = T // NW
H2 = H // 2
BTS = 512
MASK_HI = -65536


def _pack_bf16(x):
    lo = x[:, :H2].astype(jnp.bfloat16).astype(jnp.float32)
    hi = x[:, H2:].astype(jnp.bfloat16).astype(jnp.float32)
    lob = lax.bitcast_convert_type(lo, jnp.int32)
    hib = lax.bitcast_convert_type(hi, jnp.int32)
    return jnp.bitwise_or(jnp.bitwise_and(hib, MASK_HI),
                          jnp.bitwise_and(lax.shift_right_logical(lob, 16),
                                          65535))


def _unpack_bf16(p):
    lo = lax.bitcast_convert_type(lax.shift_left(p, 16),
                                  jnp.float32).astype(jnp.bfloat16)
    hi = lax.bitcast_convert_type(jnp.bitwise_and(p, MASK_HI),
                                  jnp.float32).astype(jnp.bfloat16)
    return lo, hi


def _router_body(x_ref, gw_ref, bias_ref, ids_ref, wts_ref, xp_ref, cnt_ref):
    xp_ref[...] = _pack_bf16(x_ref[...])
    x = x_ref[...].astype(jnp.bfloat16)
    gw = gw_ref[...].astype(jnp.bfloat16)
    logits = lax.dot_general(
        x, gw, (((1,), (0,)), ((), ())),
        preferred_element_type=jnp.float32)
    scores = jax.nn.sigmoid(logits)
    sc = scores + bias_ref[0:1, :]
    bt = sc.shape[0]
    iota8 = lax.broadcasted_iota(jnp.int32, (bt, E), 1)
    gi = iota8 // (E // N_GROUP)

    r8 = lax.broadcasted_iota(jnp.int32, (E, E), 0)
    c8 = lax.broadcasted_iota(jnp.int32, (E, E), 1)
    pairm = (r8 // 2 == c8 // 2).astype(jnp.float32)
    gs8 = lax.dot_general(
        sc, pairm, (((1,), (0,)), ((), ())),
        precision=lax.Precision.HIGHEST,
        preferred_element_type=jnp.float32)

    m1 = jnp.max(gs8, axis=1, keepdims=True)
    g1 = jnp.min(jnp.where(gs8 == m1, gi, N_GROUP), axis=1, keepdims=True)
    gs8b = jnp.where(gi == g1, -jnp.inf, gs8)
    m2 = jnp.max(gs8b, axis=1, keepdims=True)
    g2 = jnp.min(jnp.where(gs8b == m2, gi, N_GROUP), axis=1, keepdims=True)
    sel = (gi == g1) | (gi == g2)

    tmp = jnp.where(sel, sc, 0.0)
    t1 = jnp.max(tmp, axis=1, keepdims=True)
    e1 = jnp.min(jnp.where(tmp == t1, iota8, E), axis=1, keepdims=True)
    tmp2 = jnp.where(iota8 == e1, -jnp.inf, tmp)
    t2 = jnp.max(tmp2, axis=1, keepdims=True)
    e2 = jnp.min(jnp.where(tmp2 == t2, iota8, E), axis=1, keepdims=True)

    w1 = jnp.sum(jnp.where(iota8 == e1, scores, 0.0), axis=1, keepdims=True)
    w2 = jnp.sum(jnp.where(iota8 == e2, scores, 0.0), axis=1, keepdims=True)
    denom = w1 + w2 + 1e-20
    ids_ref[...] = jnp.concatenate([e1, e2], axis=1)
    wts_ref[...] = jnp.concatenate([w1, w2], axis=1) * (SCALE / denom)

    l64 = lax.broadcasted_iota(jnp.int32, (bt, 64), 1)
    r64 = lax.broadcasted_iota(jnp.int32, (bt, 64), 0)
    sel_e = jnp.where(l64 // 32 == 0, e1, e2)
    m = (sel_e == l64 % 16) & (r64 // PW == (l64 // 16) % 2)
    cnt_ref[pl.ds(pl.program_id(0), 1), :] = jnp.sum(
        m.astype(jnp.int32), axis=0, keepdims=True)


def _dispatch_body(ids_hbm, cnts_hbm, hid_hbm, xs_hbm, slot_hbm, te_hbm,
                   nact_hbm, ids_v, slots_v, idxc_v, allcnt_v, rowbuf_v,
                   tev_v, nactv_v, sem, lsem):
    ci = lax.axis_index("c")
    si = lax.axis_index("s")
    wid = si * 2 + ci
    base_p = (wid % 4) // 2 * T + wid // 4 * BT + wid % 2 * PW
    iota = lax.iota(jnp.int32, 16)

    pltpu.sync_copy(ids_hbm.at[pl.ds(base_p, PW)], ids_v)
    pltpu.sync_copy(cnts_hbm, allcnt_v)

    tot = jnp.zeros(16, jnp.int32)
    pref = jnp.zeros(16, jnp.int32)
    for tt in range(8):
        for jj in range(4):
            ww = tt * 4 + jj
            row = allcnt_v[tt, pl.ds(jj * 16, 16)]
            tot = tot + row
            pref = pref + row * (ww < wid).astype(jnp.int32)
    totpad = (tot + (BT - 1)) // BT * BT
    incl = plsc.cumsum(totpad)
    excl = incl - totpad
    basev = excl + pref
    base_sc = [jnp.sum(basev * (iota == e).astype(jnp.int32)) for e in range(E)]
    incl_sc = [jnp.sum(incl * (iota == e).astype(jnp.int32)) for e in range(E)]
    total_pad = incl_sc[E - 1]

    run = list(base_sc)
    for v in range(PW // 16):
        vec = ids_v[pl.ds(v * 16, 16)]
        slot_vec = jnp.zeros(16, jnp.int32)
        for e in range(E):
            m = vec == e
            mi = m.astype(jnp.int32)
            ranks = plsc.cumsum(mi) - 1
            slot_vec = jnp.where(m, run[e] + ranks, slot_vec)
            run[e] = run[e] + jnp.sum(mi)
        slots_v[pl.ds(v * 16, 16)] = slot_vec
    pltpu.sync_copy(slots_v, slot_hbm.at[pl.ds(base_p, PW)])

    @pl.when(wid == 0)
    def _():
        for half in range(2):
            ivec = (iota + 16 * half) * BT
            acc = jnp.zeros(16, jnp.int32)
            for e in range(E):
                acc = acc + (ivec >= incl_sc[e]).astype(jnp.int32)
            tev_v[pl.ds(16 * half, 16)] = jnp.minimum(acc, E - 1)
        nactv_v[...] = (iota == 0).astype(jnp.int32) * (total_pad // BT)
        pltpu.sync_copy(tev_v, te_hbm)
        pltpu.sync_copy(nactv_v, nact_hbm)

    tok0 = base_p % T
    nch = PW // GCH
    lcp = [None, None]
    scp = [None, None]
    lcp[0] = pltpu.async_copy(hid_hbm.at[pl.ds(tok0, GCH)], rowbuf_v.at[0], lsem)
    for ch in range(nch):
        b = ch % 2
        lcp[b].wait()
        for i in range(GCH // 16):
            idxc_v[b, pl.ds(i * 16, 16)] = slots_v[pl.ds(ch * GCH + i * 16, 16)]
        scp[b] = pltpu.async_copy(rowbuf_v.at[b], xs_hbm.at[idxc_v.at[b]], sem)
        if ch + 1 < nch:
            if scp[1 - b] is not None:
                scp[1 - b].wait()
            lcp[1 - b] = pltpu.async_copy(
                hid_hbm.at[pl.ds(tok0 + (ch + 1) * GCH, GCH)],
                rowbuf_v.at[1 - b], lsem)
    for b in range(2):
        if scp[b] is None:
            continue
        scp[b].wait()


def _gemm_body(te_ref, nact_ref, xs_ref, wgu_ref, wd_ref, ys_ref, xbf_ref):
    i = pl.program_id(0)

    @pl.when(i < nact_ref[0])
    def _():
        xlo, xhi = _unpack_bf16(xs_ref[...])
        xbf_ref[:, :H2] = xlo
        xbf_ref[:, H2:] = xhi
        h1 = lax.dot_general(
            xbf_ref[...], wgu_ref[0], (((1,), (0,)), ((), ())),
            preferred_element_type=jnp.float32)
        g = h1[:, :I]
        u = h1[:, I:]
        act = (jax.nn.silu(g) * u).astype(jnp.bfloat16)
        ys_ref[...] = _pack_bf16(lax.dot_general(
            act, wd_ref[0], (((1,), (0,)), ((), ())),
            preferred_element_type=jnp.float32))


def _shared_body(xp_in_ref, wgu_ref, wd_ref, y0_ref, y1_ref, wts_ref, out_ref,
                 xbf_ref):
    xlo, xhi = _unpack_bf16(xp_in_ref[...])
    xbf_ref[:, :H2] = xlo
    xbf_ref[:, H2:] = xhi
    gu = lax.dot_general(
        xbf_ref[...], wgu_ref[...], (((1,), (0,)), ((), ())),
        preferred_element_type=jnp.float32)
    g = gu[:, :ISH]
    u = gu[:, ISH:]
    act = (jax.nn.silu(g) * u).astype(jnp.bfloat16)
    s = lax.dot_general(
        act, wd_ref[...], (((1,), (0,)), ((), ())),
        preferred_element_type=jnp.float32)
    wts = wts_ref[...]
    iota2 = lax.broadcasted_iota(jnp.int32, (BTS, TOPK), 1)
    w0 = jnp.sum(jnp.where(iota2 == 0, wts, 0.0), axis=1, keepdims=True)
    w1 = jnp.sum(jnp.where(iota2 == 1, wts, 0.0), axis=1, keepdims=True)
    y0lo, y0hi = _unpack_bf16(y0_ref[...])
    y1lo, y1hi = _unpack_bf16(y1_ref[...])
    out_ref[:, :H2] = (s[:, :H2] + w0 * y0lo.astype(jnp.float32)
                       + w1 * y1lo.astype(jnp.float32))
    out_ref[:, H2:] = (s[:, H2:] + w0 * y0hi.astype(jnp.float32)
                       + w1 * y1hi.astype(jnp.float32))


def _gather_body(ys_hbm, slot_hbm, y0_hbm, y1_hbm, slots_v, idxc_v, rows_v,
                 sem, lsem):
    ci = lax.axis_index("c")
    si = lax.axis_index("s")
    wid = si * 2 + ci
    pltpu.sync_copy(slot_hbm.at[pl.ds(wid * TPW, TPW)],
                    slots_v.at[pl.ds(0, TPW)])
    pltpu.sync_copy(slot_hbm.at[pl.ds(T + wid * TPW, TPW)],
                    slots_v.at[pl.ds(TPW, TPW)])
    nch = TPW // 16
    steps = [(k, ch) for k in range(2) for ch in range(nch)]
    gcp = [None, None]
    wcp = [None, None]

    def start_gather(i, b):
        k, ch = steps[i]
        idxc_v[b, pl.ds(0, 16)] = slots_v[pl.ds(k * TPW + ch * 16, 16)]
        return pltpu.async_copy(ys_hbm.at[idxc_v.at[b]], rows_v.at[b], lsem)

    gcp[0] = start_gather(0, 0)
    for i in range(len(steps)):
        b = i % 2
        k, ch = steps[i]
        gcp[b].wait()
        dst = y0_hbm if k == 0 else y1_hbm
        wcp[b] = pltpu.async_copy(
            rows_v.at[b],
            dst.at[pl.ds(wid * TPW + ch * 16, 16)],
            sem)
        if i + 1 < len(steps):
            if wcp[1 - b] is not None:
                wcp[1 - b].wait()
            gcp[1 - b] = start_gather(i + 1, 1 - b)
    for b in range(2):
        if wcp[b] is None:
            continue
        wcp[b].wait()


def kernel(hidden_states, gate_w, e_score_correction_bias, w_gate_up, w_down,
           shared_gate_up, shared_down):
    bias8 = jnp.broadcast_to(e_score_correction_bias[None, :], (8, E))

    ids, wts, xp, cnts = pl.pallas_call(
        _router_body,
        grid=(T // BT,),
        in_specs=[
            pl.BlockSpec((BT, H), lambda t: (t, 0)),
            pl.BlockSpec((H, E), lambda t: (0, 0)),
            pl.BlockSpec((8, E), lambda t: (0, 0)),
        ],
        out_specs=[
            pl.BlockSpec((BT, TOPK), lambda t: (t, 0)),
            pl.BlockSpec((BT, TOPK), lambda t: (t, 0)),
            pl.BlockSpec((BT, H2), lambda t: (t, 0)),
            pl.BlockSpec((8, 64), lambda t: (0, 0)),
        ],
        out_shape=[
            jax.ShapeDtypeStruct((T, TOPK), jnp.int32),
            jax.ShapeDtypeStruct((T, TOPK), jnp.float32),
            jax.ShapeDtypeStruct((T, H2), jnp.int32),
            jax.ShapeDtypeStruct((8, 64), jnp.int32),
        ],
    )(hidden_states, gate_w, bias8)

    ids_flat = ids.T.reshape(T * TOPK)

    xs, slot_arr, te, nact = pl.kernel(
        _dispatch_body,
        mesh=plsc.VectorSubcoreMesh(core_axis_name="c", subcore_axis_name="s"),
        compiler_params=pltpu.CompilerParams(needs_layout_passes=False),
        out_type=(
            jax.ShapeDtypeStruct((R, H2), jnp.int32),
            jax.ShapeDtypeStruct((T * TOPK,), jnp.int32),
            jax.ShapeDtypeStruct((NW,), jnp.int32),
            jax.ShapeDtypeStruct((16,), jnp.int32),
        ),
        scratch_types=[
            pltpu.VMEM((PW,), jnp.int32),
            pltpu.VMEM((PW,), jnp.int32),
            pltpu.VMEM((2, GCH), jnp.int32),
            pltpu.VMEM((8, 64), jnp.int32),
            pltpu.VMEM((2, GCH, H2), jnp.int32),
            pltpu.VMEM((NW,), jnp.int32),
            pltpu.VMEM((16,), jnp.int32),
            pltpu.SemaphoreType.DMA,
            pltpu.SemaphoreType.DMA,
        ],
    )(ids_flat, cnts, xp)

    wgu_bf = w_gate_up.astype(jnp.bfloat16)
    wd_bf = w_down.astype(jnp.bfloat16)
    ys = pl.pallas_call(
        _gemm_body,
        grid_spec=pltpu.PrefetchScalarGridSpec(
            num_scalar_prefetch=2,
            grid=(NT,),
            in_specs=[
                pl.BlockSpec((BT, H2), lambda i, te, na: (i, 0)),
                pl.BlockSpec((1, H, 2 * I), lambda i, te, na: (te[i], 0, 0)),
                pl.BlockSpec((1, I, H), lambda i, te, na: (te[i], 0, 0)),
            ],
            out_specs=pl.BlockSpec((BT, H2), lambda i, te, na: (i, 0)),
            scratch_shapes=[pltpu.VMEM((BT, H), jnp.bfloat16)],
        ),
        out_shape=jax.ShapeDtypeStruct((R, H2), jnp.int32),
    )(te, nact, xs, wgu_bf, wd_bf)

    y0, y1 = pl.kernel(
        _gather_body,
        mesh=plsc.VectorSubcoreMesh(core_axis_name="c", subcore_axis_name="s"),
        compiler_params=pltpu.CompilerParams(needs_layout_passes=False),
        out_type=(
            jax.ShapeDtypeStruct((T, H2), jnp.int32),
            jax.ShapeDtypeStruct((T, H2), jnp.int32),
        ),
        scratch_types=[
            pltpu.VMEM((PW,), jnp.int32),
            pltpu.VMEM((2, 16), jnp.int32),
            pltpu.VMEM((2, 16, H2), jnp.int32),
            pltpu.SemaphoreType.DMA,
            pltpu.SemaphoreType.DMA,
        ],
    )(ys, slot_arr)

    sgu_bf = shared_gate_up.astype(jnp.bfloat16)
    sd_bf = shared_down.astype(jnp.bfloat16)
    out = pl.pallas_call(
        _shared_body,
        grid=(T // BTS,),
        in_specs=[
            pl.BlockSpec((BTS, H2), lambda t: (t, 0)),
            pl.BlockSpec((H, 2 * ISH), lambda t: (0, 0)),
            pl.BlockSpec((ISH, H), lambda t: (0, 0)),
            pl.BlockSpec((BTS, H2), lambda t: (t, 0)),
            pl.BlockSpec((BTS, H2), lambda t: (t, 0)),
            pl.BlockSpec((BTS, TOPK), lambda t: (t, 0)),
        ],
        out_specs=pl.BlockSpec((BTS, H), lambda t: (t, 0)),
        out_shape=jax.ShapeDtypeStruct((T, H), jnp.float32),
        scratch_shapes=[pltpu.VMEM((BTS, H), jnp.bfloat16)],
    )(xp, sgu_bf, sd_bf, y0, y1, wts)
    return out
